# baseline (speedup 1.0000x reference)
import jax
import jax.numpy as jnp
from jax.experimental import pallas as pl
from jax.experimental.pallas import tpu as pltpu

_NCHUNK = 8
_ROWS = 4096 // _NCHUNK


def _copy_kernel(x_ref, o_ref, buf, sem_in, sem_out):
    ins = [
        pltpu.make_async_copy(
            x_ref.at[pl.ds(i * _ROWS, _ROWS)], buf.at[i], sem_in.at[i]
        )
        for i in range(_NCHUNK)
    ]
    outs = [
        pltpu.make_async_copy(
            buf.at[i], o_ref.at[pl.ds(i * _ROWS, _ROWS)], sem_out.at[i]
        )
        for i in range(_NCHUNK)
    ]
    for c in ins:
        c.start()
    for i in range(_NCHUNK):
        ins[i].wait()
        outs[i].start()
    for c in outs:
        c.wait()


def kernel(x, embed):
    del embed
    rows, cols = x.shape
    return pl.pallas_call(
        _copy_kernel,
        out_shape=jax.ShapeDtypeStruct(x.shape, x.dtype),
        in_specs=[pl.BlockSpec(memory_space=pl.ANY)],
        out_specs=pl.BlockSpec(memory_space=pl.ANY),
        scratch_shapes=[
            pltpu.VMEM((_NCHUNK, _ROWS, cols), x.dtype),
            pltpu.SemaphoreType.DMA((_NCHUNK,)),
            pltpu.SemaphoreType.DMA((_NCHUNK,)),
        ],
    )(x)
